# baseline (speedup 1.0000x reference)
import functools
import jax
import jax.numpy as jnp
from jax import lax
from jax.experimental import pallas as pl
from jax.experimental.pallas import tpu as pltpu, tpu_sc as plsc

SEQ = 2048
D = 2048
CHUNK_ROWS = 16
NBUF = 3
LANES = 16


def _make_sc_kernel(nc, ns):
    nw = nc * ns
    rows_per_worker = SEQ // nw
    n_chunks = rows_per_worker // CHUNK_ROWS
    chunk_elems = CHUNK_ROWS * D

    mesh = plsc.VectorSubcoreMesh(core_axis_name="c", subcore_axis_name="s")

    @functools.partial(
        pl.kernel,
        out_type=(
            jax.ShapeDtypeStruct((SEQ * D,), jnp.float32),
            jax.ShapeDtypeStruct((SEQ * D,), jnp.float32),
        ),
        mesh=mesh,
        scratch_types=(
            pltpu.VMEM((D,), jnp.float32),
            pltpu.VMEM((D,), jnp.float32),
            [pltpu.VMEM((chunk_elems,), jnp.float32) for _ in range(NBUF)],
            [pltpu.SemaphoreType.DMA for _ in range(NBUF)],
            [pltpu.SemaphoreType.DMA for _ in range(NBUF)],
        ),
    )
    def sc_kernel(ft_hbm, fi_hbm, ttab_hbm, itab_hbm, tout_hbm, iout_hbm,
                  ft_v, fi_v, bufs, in_sems, out_sems):
        wid = lax.axis_index("s") * nc + lax.axis_index("c")
        base_row = wid * rows_per_worker

        pltpu.sync_copy(ft_hbm, ft_v)
        pltpu.sync_copy(fi_hbm, fi_v)

        tasks = []
        for feat_v, tab, out in ((ft_v, ttab_hbm, tout_hbm),
                                 (fi_v, itab_hbm, iout_hbm)):
            for ci in range(n_chunks):
                off = (base_row + ci * CHUNK_ROWS) * D
                tasks.append((feat_v, tab, out, off))
        nk = len(tasks)

        def start_in(k):
            _, tab, _, off = tasks[k]
            return pltpu.async_copy(
                tab.at[pl.ds(off, chunk_elems)], bufs[k % NBUF],
                in_sems[k % NBUF])

        def start_out(k):
            _, _, out, off = tasks[k]
            return pltpu.async_copy(
                bufs[k % NBUF], out.at[pl.ds(off, chunk_elems)],
                out_sems[k % NBUF])

        GROUP = 32
        n_groups = D // (GROUP * LANES)

        def compute(k):
            feat_v = tasks[k][0]
            buf = bufs[k % NBUF]
            for g in range(n_groups):
                base_col = g * GROUP * LANES
                fj = [feat_v[pl.ds(base_col + c * LANES, LANES)]
                      for c in range(GROUP)]

                @plsc.parallel_loop(0, CHUNK_ROWS, step=1)
                def rbody(r):
                    row = r * D + base_col
                    for c in range(GROUP):
                        sl = pl.ds(row + c * LANES, LANES)
                        buf[sl] = buf[sl] + fj[c]

        in_fly = {0: start_in(0), 1: start_in(1)}
        out_fly = {}
        for k in range(nk):
            nxt = k + 2
            if nxt < nk:
                prev = nxt - NBUF
                if prev >= 0:
                    out_fly[prev].wait()
                in_fly[nxt] = start_in(nxt)
            in_fly[k].wait()
            compute(k)
            out_fly[k] = start_out(k)
        for k in range(max(0, nk - NBUF), nk):
            out_fly[k].wait()

    return sc_kernel


def kernel(text, image, pos_table, text_pos_table, image_pos_table):
    del pos_table
    info = plsc.get_sparse_core_info()
    sc_k = _make_sc_kernel(info.num_cores, info.num_subcores)

    ft = text.reshape(-1)
    fi = image.reshape(-1)
    ttab = text_pos_table[:SEQ].reshape(-1)
    itab = image_pos_table[:SEQ].reshape(-1)

    tout, iout = sc_k(ft, fi, ttab, itab)
    return (tout.reshape(1, SEQ, D), iout.reshape(1, SEQ, D))


# speedup vs baseline: 2.4871x; 2.4871x over previous
import functools
import jax
import jax.numpy as jnp
from jax import lax
from jax.experimental import pallas as pl
from jax.experimental.pallas import tpu as pltpu, tpu_sc as plsc

SEQ = 2048
D = 2048
CHUNK_ROWS = 16
NBUF = 3
LANES = 16


def _make_sc_kernel(nc, ns):
    nw = nc * ns
    rows_per_worker = SEQ // nw
    n_chunks = rows_per_worker // CHUNK_ROWS

    mesh = plsc.VectorSubcoreMesh(core_axis_name="c", subcore_axis_name="s")

    @functools.partial(
        pl.kernel,
        out_type=(
            jax.ShapeDtypeStruct((SEQ, D), jnp.float32),
            jax.ShapeDtypeStruct((SEQ, D), jnp.float32),
        ),
        mesh=mesh,
        scratch_types=(
            pltpu.VMEM((D,), jnp.float32),
            pltpu.VMEM((D,), jnp.float32),
            [pltpu.VMEM((CHUNK_ROWS, D), jnp.float32) for _ in range(NBUF)],
            [pltpu.SemaphoreType.DMA for _ in range(NBUF)],
            [pltpu.SemaphoreType.DMA for _ in range(NBUF)],
        ),
    )
    def sc_kernel(ft_hbm, fi_hbm, ttab_hbm, itab_hbm, tout_hbm, iout_hbm,
                  ft_v, fi_v, bufs, in_sems, out_sems):
        wid = lax.axis_index("s") * nc + lax.axis_index("c")
        base_row = wid * rows_per_worker

        pltpu.sync_copy(ft_hbm, ft_v)
        pltpu.sync_copy(fi_hbm, fi_v)

        tasks = []
        for feat_v, tab, out in ((ft_v, ttab_hbm, tout_hbm),
                                 (fi_v, itab_hbm, iout_hbm)):
            for ci in range(n_chunks):
                tasks.append((feat_v, tab, out, base_row + ci * CHUNK_ROWS))
        nk = len(tasks)

        def start_in(k):
            _, tab, _, row0 = tasks[k]
            return pltpu.async_copy(
                tab.at[pl.ds(row0, CHUNK_ROWS)], bufs[k % NBUF],
                in_sems[k % NBUF])

        def start_out(k):
            _, _, out, row0 = tasks[k]
            return pltpu.async_copy(
                bufs[k % NBUF], out.at[pl.ds(row0, CHUNK_ROWS)],
                out_sems[k % NBUF])

        GROUP = 32
        n_groups = D // (GROUP * LANES)

        def compute(k):
            feat_v = tasks[k][0]
            buf = bufs[k % NBUF]
            for g in range(n_groups):
                base_col = g * GROUP * LANES
                fj = [feat_v[pl.ds(base_col + c * LANES, LANES)]
                      for c in range(GROUP)]

                @plsc.parallel_loop(0, CHUNK_ROWS, step=1)
                def rbody(r):
                    for c in range(GROUP):
                        sl = pl.ds(base_col + c * LANES, LANES)
                        buf[r, sl] = buf[r, sl] + fj[c]

        in_fly = {0: start_in(0), 1: start_in(1)}
        out_fly = {}
        for k in range(nk):
            nxt = k + 2
            if nxt < nk:
                prev = nxt - NBUF
                if prev >= 0:
                    out_fly[prev].wait()
                in_fly[nxt] = start_in(nxt)
            in_fly[k].wait()
            compute(k)
            out_fly[k] = start_out(k)
        for k in range(max(0, nk - NBUF), nk):
            out_fly[k].wait()

    return sc_kernel


def kernel(text, image, pos_table, text_pos_table, image_pos_table):
    del pos_table
    info = plsc.get_sparse_core_info()
    sc_k = _make_sc_kernel(info.num_cores, info.num_subcores)

    tout, iout = sc_k(text.reshape(-1), image.reshape(-1),
                      text_pos_table, image_pos_table)
    return (tout[None], iout[None])
